# Initial kernel scaffold; baseline (speedup 1.0000x reference)
import jax
import jax.numpy as jnp
from jax.experimental import pallas as pl


def kernel(x, edge_index, edge_attr, u, batch, W1, b1, W2, b2):
    raise NotImplementedError("write your pallas kernel here")


# speedup vs baseline: 8.0454x; 8.0454x over previous
import functools

import jax
import jax.numpy as jnp
from jax import lax
from jax.experimental import pallas as pl
from jax.experimental.pallas import tpu as pltpu
from jax.experimental.pallas import tpu_sc as plsc

N_NODES = 10000
N_EDGES = 320000
D_FEAT = 128
D_EDGE = 16
N_GRAPHS = 8
LATENT = 128

_BN = 1000


def _tables_body(x_ref, bt_ref, u_ref, wa_ref, wb_ref, wu_ref, b1_ref,
                 a_ref, b_ref):
    x = x_ref[...]
    uw = jnp.dot(u_ref[...], wu_ref[...], preferred_element_type=jnp.float32)
    oh = (bt_ref[...] == lax.broadcasted_iota(jnp.int32, (_BN, N_GRAPHS), 1)
          ).astype(jnp.float32)
    a_ref[...] = (jnp.dot(x, wa_ref[...], preferred_element_type=jnp.float32)
                  + jnp.dot(oh, uw, preferred_element_type=jnp.float32)
                  + b1_ref[...])
    b_ref[...] = jnp.dot(x, wb_ref[...], preferred_element_type=jnp.float32)


def _node_tables(x, batch2d, u, wa, wb, wu, b1):
    grid = (N_NODES // _BN,)
    return pl.pallas_call(
        _tables_body,
        grid=grid,
        in_specs=[
            pl.BlockSpec((_BN, D_FEAT), lambda i: (i, 0)),
            pl.BlockSpec((_BN, 1), lambda i: (i, 0)),
            pl.BlockSpec((N_GRAPHS, D_FEAT), lambda i: (0, 0)),
            pl.BlockSpec((D_FEAT, LATENT), lambda i: (0, 0)),
            pl.BlockSpec((D_FEAT, LATENT), lambda i: (0, 0)),
            pl.BlockSpec((D_FEAT, LATENT), lambda i: (0, 0)),
            pl.BlockSpec((1, LATENT), lambda i: (0, 0)),
        ],
        out_specs=[
            pl.BlockSpec((_BN, LATENT), lambda i: (i, 0)),
            pl.BlockSpec((_BN, LATENT), lambda i: (i, 0)),
        ],
        out_shape=[
            jax.ShapeDtypeStruct((N_NODES, LATENT), jnp.float32),
            jax.ShapeDtypeStruct((N_NODES, LATENT), jnp.float32),
        ],
    )(x, batch2d, u, wa, wb, wu, b1)


_NC = 2
_NS = 16
_NW = _NC * _NS
_PER_W = N_EDGES // _NW
_CB = 80
_NCHUNK = _PER_W // _CB

@functools.cache
def _build_gather():
    mesh = plsc.VectorSubcoreMesh(core_axis_name="c", subcore_axis_name="s",
                                  num_cores=_NC, num_subcores=_NS)

    @functools.partial(
        pl.kernel,
        mesh=mesh,
        out_type=[
            jax.ShapeDtypeStruct((N_EDGES, LATENT), jnp.float32),
            jax.ShapeDtypeStruct((N_EDGES, LATENT), jnp.float32),
        ],
        scratch_types=[
            pltpu.VMEM((_NCHUNK, _CB), jnp.int32),
            pltpu.VMEM((_NCHUNK, _CB), jnp.int32),
            pltpu.VMEM((_CB, LATENT), jnp.float32),
            pltpu.VMEM((_CB, LATENT), jnp.float32),
            pltpu.SemaphoreType.DMA,
            pltpu.SemaphoreType.DMA,
        ],
    )
    def _gather_tables(a_hbm, b_hbm, src_hbm, dst_hbm, ga_hbm, gb_hbm,
                       si_v, di_v, bufa, bufb, sema, semb):
        wid = lax.axis_index("s") * _NC + lax.axis_index("c")
        base = wid * _PER_W
        pltpu.sync_copy(src_hbm.at[wid], si_v)
        pltpu.sync_copy(dst_hbm.at[wid], di_v)

        def body(c, carry):
            ca = pltpu.async_copy(a_hbm.at[si_v.at[c]], bufa, sema)
            cb = pltpu.async_copy(b_hbm.at[di_v.at[c]], bufb, semb)
            ca.wait()
            cb.wait()
            row = base + c * _CB
            pltpu.sync_copy(bufa, ga_hbm.at[pl.ds(row, _CB)])
            pltpu.sync_copy(bufb, gb_hbm.at[pl.ds(row, _CB)])
            return carry

        lax.fori_loop(0, _NCHUNK, body, 0)

    return _gather_tables


_BE = 2000


def _mlp_body(ga_ref, gb_ref, ea_ref, we_ref, w2_ref, b2_ref, o_ref):
    h = (ga_ref[...] + gb_ref[...]
         + jnp.dot(ea_ref[...], we_ref[...],
                   preferred_element_type=jnp.float32))
    h = jnp.maximum(h, 0.0)
    o_ref[...] = jnp.maximum(
        jnp.dot(h, w2_ref[...], preferred_element_type=jnp.float32)
        + b2_ref[...], 0.0)


def _edge_mlp(ga, gb, ea, we, w2, b2):
    grid = (N_EDGES // _BE,)
    return pl.pallas_call(
        _mlp_body,
        grid=grid,
        in_specs=[
            pl.BlockSpec((_BE, LATENT), lambda i: (i, 0)),
            pl.BlockSpec((_BE, LATENT), lambda i: (i, 0)),
            pl.BlockSpec((_BE, D_EDGE), lambda i: (i, 0)),
            pl.BlockSpec((D_EDGE, LATENT), lambda i: (0, 0)),
            pl.BlockSpec((LATENT, LATENT), lambda i: (0, 0)),
            pl.BlockSpec((1, LATENT), lambda i: (0, 0)),
        ],
        out_specs=pl.BlockSpec((_BE, LATENT), lambda i: (i, 0)),
        out_shape=jax.ShapeDtypeStruct((N_EDGES, LATENT), jnp.float32),
    )(ga, gb, ea, we, w2, b2)


def kernel(x, edge_index, edge_attr, u, batch, W1, b1, W2, b2):
    src = edge_index[0].astype(jnp.int32).reshape(_NW, _NCHUNK, _CB)
    dst = edge_index[1].astype(jnp.int32).reshape(_NW, _NCHUNK, _CB)
    batch2d = batch.astype(jnp.int32).reshape(N_NODES, 1)
    wa = W1[0:D_FEAT]
    wb = W1[D_FEAT:2 * D_FEAT]
    we = W1[2 * D_FEAT:2 * D_FEAT + D_EDGE]
    wu = W1[2 * D_FEAT + D_EDGE:]
    a_tab, b_tab = _node_tables(x, batch2d, u, wa, wb, wu,
                                b1.reshape(1, LATENT))
    ga, gb = _build_gather()(a_tab, b_tab, src, dst)
    return _edge_mlp(ga, gb, edge_attr, we, W2, b2.reshape(1, LATENT))


# speedup vs baseline: 8.5822x; 1.0667x over previous
import functools

import jax
import jax.numpy as jnp
from jax import lax
from jax.experimental import pallas as pl
from jax.experimental.pallas import tpu as pltpu
from jax.experimental.pallas import tpu_sc as plsc

N_NODES = 10000
N_EDGES = 320000
D_FEAT = 128
D_EDGE = 16
N_GRAPHS = 8
LATENT = 128

_BN = 1000


def _tables_body(x_ref, bt_ref, u_ref, wa_ref, wb_ref, wu_ref, b1_ref,
                 a_ref, b_ref):
    x = x_ref[...]
    uw = jnp.dot(u_ref[...], wu_ref[...], preferred_element_type=jnp.float32)
    oh = (bt_ref[...] == lax.broadcasted_iota(jnp.int32, (_BN, N_GRAPHS), 1)
          ).astype(jnp.float32)
    a_ref[...] = (jnp.dot(x, wa_ref[...], preferred_element_type=jnp.float32)
                  + jnp.dot(oh, uw, preferred_element_type=jnp.float32)
                  + b1_ref[...])
    b_ref[...] = jnp.dot(x, wb_ref[...], preferred_element_type=jnp.float32)


def _node_tables(x, batch2d, u, wa, wb, wu, b1):
    grid = (N_NODES // _BN,)
    return pl.pallas_call(
        _tables_body,
        grid=grid,
        in_specs=[
            pl.BlockSpec((_BN, D_FEAT), lambda i: (i, 0)),
            pl.BlockSpec((_BN, 1), lambda i: (i, 0)),
            pl.BlockSpec((N_GRAPHS, D_FEAT), lambda i: (0, 0)),
            pl.BlockSpec((D_FEAT, LATENT), lambda i: (0, 0)),
            pl.BlockSpec((D_FEAT, LATENT), lambda i: (0, 0)),
            pl.BlockSpec((D_FEAT, LATENT), lambda i: (0, 0)),
            pl.BlockSpec((1, LATENT), lambda i: (0, 0)),
        ],
        out_specs=[
            pl.BlockSpec((_BN, LATENT), lambda i: (i, 0)),
            pl.BlockSpec((_BN, LATENT), lambda i: (i, 0)),
        ],
        out_shape=[
            jax.ShapeDtypeStruct((N_NODES, LATENT), jnp.float32),
            jax.ShapeDtypeStruct((N_NODES, LATENT), jnp.float32),
        ],
    )(x, batch2d, u, wa, wb, wu, b1)


_NC = 2
_NS = 16
_NW = _NC * _NS
_CB = 128
_NCHUNK = 80
_PER_W = _NCHUNK * _CB
_E_PAD = _NW * _PER_W


@functools.cache
def _build_gather():
    mesh = plsc.VectorSubcoreMesh(core_axis_name="c", subcore_axis_name="s",
                                  num_cores=_NC, num_subcores=_NS)

    @functools.partial(
        pl.kernel,
        mesh=mesh,
        out_type=[
            jax.ShapeDtypeStruct((_E_PAD, LATENT), jnp.float32),
            jax.ShapeDtypeStruct((_E_PAD, LATENT), jnp.float32),
        ],
        scratch_types=[
            pltpu.VMEM((_NCHUNK, _CB), jnp.int32),
            pltpu.VMEM((_NCHUNK, _CB), jnp.int32),
            pltpu.VMEM((_CB, LATENT), jnp.float32),
            pltpu.VMEM((_CB, LATENT), jnp.float32),
            pltpu.VMEM((_CB, LATENT), jnp.float32),
            pltpu.VMEM((_CB, LATENT), jnp.float32),
            pltpu.SemaphoreType.DMA,
            pltpu.SemaphoreType.DMA,
        ],
    )
    def _gather_tables(a_hbm, b_hbm, src_hbm, dst_hbm, ga_hbm, gb_hbm,
                       si_v, di_v, bufa0, bufb0, bufa1, bufb1, s0, s1):
        wid = lax.axis_index("s") * _NC + lax.axis_index("c")
        base = wid * _PER_W
        pltpu.sync_copy(src_hbm.at[wid], si_v)
        pltpu.sync_copy(dst_hbm.at[wid], di_v)

        def gathers(c, bufa, bufb, sem):
            ca = pltpu.async_copy(a_hbm.at[si_v.at[c]], bufa, sem)
            cb = pltpu.async_copy(b_hbm.at[di_v.at[c]], bufb, sem)
            return ca, cb

        def drain(c, bufa, bufb, sem):
            pltpu.make_async_copy(a_hbm.at[si_v.at[c]], bufa, sem).wait()
            pltpu.make_async_copy(b_hbm.at[di_v.at[c]], bufb, sem).wait()

        def writeback(c, bufa, bufb):
            row = base + c * _CB
            pltpu.sync_copy(bufa, ga_hbm.at[pl.ds(row, _CB)])
            pltpu.sync_copy(bufb, gb_hbm.at[pl.ds(row, _CB)])

        gathers(0, bufa0, bufb0, s0)

        def body(i, carry):
            c0 = 2 * i
            c1 = c0 + 1
            gathers(c1, bufa1, bufb1, s1)
            drain(c0, bufa0, bufb0, s0)
            writeback(c0, bufa0, bufb0)

            @pl.when(i < _NCHUNK // 2 - 1)
            def _():
                gathers(c0 + 2, bufa0, bufb0, s0)

            drain(c1, bufa1, bufb1, s1)
            writeback(c1, bufa1, bufb1)
            return carry

        lax.fori_loop(0, _NCHUNK // 2, body, 0)

    return _gather_tables


_BE = 2000


def _mlp_body(ga_ref, gb_ref, ea_ref, we_ref, w2_ref, b2_ref, o_ref):
    h = (ga_ref[...] + gb_ref[...]
         + jnp.dot(ea_ref[...], we_ref[...],
                   preferred_element_type=jnp.float32))
    h = jnp.maximum(h, 0.0)
    o_ref[...] = jnp.maximum(
        jnp.dot(h, w2_ref[...], preferred_element_type=jnp.float32)
        + b2_ref[...], 0.0)


def _edge_mlp(ga, gb, ea, we, w2, b2):
    grid = (N_EDGES // _BE,)
    return pl.pallas_call(
        _mlp_body,
        grid=grid,
        in_specs=[
            pl.BlockSpec((_BE, LATENT), lambda i: (i, 0)),
            pl.BlockSpec((_BE, LATENT), lambda i: (i, 0)),
            pl.BlockSpec((_BE, D_EDGE), lambda i: (i, 0)),
            pl.BlockSpec((D_EDGE, LATENT), lambda i: (0, 0)),
            pl.BlockSpec((LATENT, LATENT), lambda i: (0, 0)),
            pl.BlockSpec((1, LATENT), lambda i: (0, 0)),
        ],
        out_specs=pl.BlockSpec((_BE, LATENT), lambda i: (i, 0)),
        out_shape=jax.ShapeDtypeStruct((N_EDGES, LATENT), jnp.float32),
    )(ga, gb, ea, we, w2, b2)


def kernel(x, edge_index, edge_attr, u, batch, W1, b1, W2, b2):
    pad = (jnp.arange(_E_PAD - N_EDGES, dtype=jnp.int32) % N_NODES)
    src = jnp.concatenate([edge_index[0].astype(jnp.int32), pad]
                          ).reshape(_NW, _NCHUNK, _CB)
    dst = jnp.concatenate([edge_index[1].astype(jnp.int32), pad]
                          ).reshape(_NW, _NCHUNK, _CB)
    batch2d = batch.astype(jnp.int32).reshape(N_NODES, 1)
    wa = W1[0:D_FEAT]
    wb = W1[D_FEAT:2 * D_FEAT]
    we = W1[2 * D_FEAT:2 * D_FEAT + D_EDGE]
    wu = W1[2 * D_FEAT + D_EDGE:]
    a_tab, b_tab = _node_tables(x, batch2d, u, wa, wb, wu,
                                b1.reshape(1, LATENT))
    ga, gb = _build_gather()(a_tab, b_tab, src, dst)
    return _edge_mlp(ga, gb, edge_attr, we, W2, b2.reshape(1, LATENT))
